# Initial kernel scaffold; baseline (speedup 1.0000x reference)
import jax
import jax.numpy as jnp
from jax.experimental import pallas as pl


def kernel(table, seq_len):
    raise NotImplementedError("write your pallas kernel here")


# speedup vs baseline: 38.8232x; 38.8232x over previous
import functools

import jax
import jax.numpy as jnp
from jax import lax
from jax.experimental import pallas as pl
from jax.experimental.pallas import tpu as pltpu
from jax.experimental.pallas import tpu_sc as plsc

H = 16
S = 2048
NC = 2
NS = 16
HEADS_PER_PHASE = 4
PHASES = H // NC // HEADS_PER_PHASE
W = 15 * 128 + S
RB = 128


def _sc_expand_call(tabr):
    mesh = plsc.VectorSubcoreMesh(core_axis_name="c", subcore_axis_name="s")

    @functools.partial(
        pl.kernel,
        mesh=mesh,
        out_type=jax.ShapeDtypeStruct((H, S, S), jnp.float32),
        scratch_types=[
            pltpu.VMEM_SHARED((HEADS_PER_PHASE, RB, W), jnp.float32),
            pltpu.SemaphoreType.DMA,
        ],
    )
    def sc_expand(tabr_hbm, out_hbm, sp, sem):
        cid = lax.axis_index("c")
        sid = lax.axis_index("s")
        hs = sid // 4
        qtr = sid % 4

        for phase in range(PHASES):
            h = cid * (H // NC) + phase * HEADS_PER_PHASE + hs
            pltpu.sync_copy(
                tabr_hbm.at[h, pl.ds(32 * qtr, 32)],
                sp.at[hs, pl.ds(32 * qtr, 32)],
            )
            plsc.subcore_barrier()

            for b in range(4):
                i0 = qtr * 512 + RB * b
                q = 15 - i0 // RB
                src = sp.at[hs, :, pl.ds(RB * q, S)]
                dst = out_hbm.at[h, pl.ds(i0, RB)]
                pltpu.make_async_copy(src, dst, sem).start()
            for b in range(4):
                src = sp.at[hs, :, pl.ds(0, S)]
                dst = out_hbm.at[h, pl.ds(qtr * 512 + RB * b, RB)]
                pltpu.make_async_copy(src, dst, sem).wait()
            plsc.subcore_barrier()

    return sc_expand(tabr)


def kernel(table, seq_len):
    del seq_len
    ftf = jnp.flip(table, axis=0).T
    tabr = jnp.stack(
        [ftf[:, 127 - r: 127 - r + W] for r in range(RB)], axis=1
    )
    return _sc_expand_call(tabr)


# speedup vs baseline: 96.4266x; 2.4837x over previous
import functools

import jax
import jax.numpy as jnp
from jax import lax
from jax.experimental import pallas as pl
from jax.experimental.pallas import tpu as pltpu
from jax.experimental.pallas import tpu_sc as plsc

H = 16
S = 2048
NC = 2
NS = 16
NW = NC * NS
W = 15 * 128 + S
PHASES = 2
M_PER_TILE = 4


def _sc_expand_call(tabv):
    mesh = plsc.VectorSubcoreMesh(core_axis_name="c", subcore_axis_name="s")

    @functools.partial(
        pl.kernel,
        mesh=mesh,
        out_type=jax.ShapeDtypeStruct((H, S, S), jnp.float32),
        scratch_types=[
            pltpu.VMEM((M_PER_TILE, 8, W), jnp.float32),
            pltpu.SemaphoreType.DMA,
        ],
    )
    def sc_expand(tabv_hbm, out_hbm, vt, sem):
        cid = lax.axis_index("c")
        sid = lax.axis_index("s")
        w = sid * NC + cid
        hslot = w // 4
        mbase = M_PER_TILE * (w % 4)

        for phase in range(PHASES):
            h = phase * (H // PHASES) + hslot
            pltpu.sync_copy(tabv_hbm.at[h, pl.ds(mbase, M_PER_TILE)], vt)

            def fire(a, carry):
                for j in range(M_PER_TILE):
                    i0 = 2040 - 8 * (mbase + j) - 128 * a
                    src = vt.at[j, :, pl.ds(128 * a, S)]
                    dst = out_hbm.at[h, pl.ds(i0, 8)]
                    pltpu.make_async_copy(src, dst, sem).start()
                return carry

            lax.fori_loop(0, 16, fire, 0)

            def drain(a, carry):
                for j in range(M_PER_TILE):
                    i0 = 2040 - 8 * (mbase + j) - 128 * a
                    src = vt.at[j, :, pl.ds(0, S)]
                    dst = out_hbm.at[h, pl.ds(i0, 8)]
                    pltpu.make_async_copy(src, dst, sem).wait()
                return carry

            lax.fori_loop(0, 16, drain, 0)

    return sc_expand(tabv)


def kernel(table, seq_len):
    del seq_len
    ftf = jnp.flip(table, axis=0).T
    tabv = jnp.stack(
        [
            jnp.stack(
                [ftf[:, 8 * m + 7 - r: 8 * m + 7 - r + W] for r in range(8)],
                axis=1,
            )
            for m in range(16)
        ],
        axis=1,
    )
    return _sc_expand_call(tabv)
